# baseline (speedup 1.0000x reference)
import functools
import jax
import jax.numpy as jnp
from jax import lax
from jax.experimental import pallas as pl
from jax.experimental.pallas import tpu as pltpu
from jax.experimental.pallas import tpu_sc as plsc

N_TOK = 2048
D = 1024
POOL = 100000
HID = 256
MIN_P = 32
MAX_P = 128

TOK_TILE = 256
POOL_BLK = 2048
POOL_PAD = 100352
NW = 32
ROWS_PER_W = (N_TOK * MAX_P) // NW
CH = 64
CT = 16


def _router_kernel(x_ref, W1_ref, b1_ref, W2_ref, b2_ref, Ws1_ref, bs1_ref,
                   h2_ref, bud_ref):
    x = x_ref[...]
    h = jnp.maximum(x @ W1_ref[...] + b1_ref[...], 0.0)
    c = jax.nn.sigmoid(h @ W2_ref[...] + b2_ref[...])
    raw = MIN_P + (MAX_P - MIN_P) * (c * c)
    bud_ref[...] = jnp.round(jnp.clip(raw, MIN_P, MAX_P))
    h2_ref[...] = jnp.maximum(x @ Ws1_ref[...] + bs1_ref[...], 0.0)


def _router(x, W1, b1, W2, b2, Ws1, bs1):
    nt = N_TOK // TOK_TILE
    return pl.pallas_call(
        _router_kernel,
        grid=(nt,),
        in_specs=[
            pl.BlockSpec((TOK_TILE, D), lambda i: (i, 0)),
            pl.BlockSpec((D, 128), lambda i: (0, 0)),
            pl.BlockSpec((1, 128), lambda i: (0, 0)),
            pl.BlockSpec((128, 1), lambda i: (0, 0)),
            pl.BlockSpec((1, 1), lambda i: (0, 0)),
            pl.BlockSpec((D, HID), lambda i: (0, 0)),
            pl.BlockSpec((1, HID), lambda i: (0, 0)),
        ],
        out_specs=[
            pl.BlockSpec((TOK_TILE, HID), lambda i: (i, 0)),
            pl.BlockSpec((TOK_TILE, 1), lambda i: (i, 0)),
        ],
        out_shape=[
            jax.ShapeDtypeStruct((N_TOK, HID), jnp.float32),
            jax.ShapeDtypeStruct((N_TOK, 1), jnp.float32),
        ],
    )(x, W1, b1.reshape(1, 128), W2, b2.reshape(1, 1), Ws1,
      bs1.reshape(1, HID))


def _scores_kernel(h2_ref, Ws2_ref, bs2_ref, out_ref):
    out_ref[...] = h2_ref[...] @ Ws2_ref[...] + bs2_ref[...]


def _scores(h2, Ws2p, bs2p):
    nt = N_TOK // TOK_TILE
    np_ = POOL_PAD // POOL_BLK
    return pl.pallas_call(
        _scores_kernel,
        grid=(nt, np_),
        in_specs=[
            pl.BlockSpec((TOK_TILE, HID), lambda i, j: (i, 0)),
            pl.BlockSpec((HID, POOL_BLK), lambda i, j: (0, j)),
            pl.BlockSpec((1, POOL_BLK), lambda i, j: (0, j)),
        ],
        out_specs=pl.BlockSpec((TOK_TILE, POOL_BLK), lambda i, j: (i, j)),
        out_shape=jax.ShapeDtypeStruct((N_TOK, POOL_PAD), jnp.float32),
    )(h2, Ws2p, bs2p)


def _sc_gather(embed, idx_flat):
    mesh = plsc.VectorSubcoreMesh(core_axis_name="c", subcore_axis_name="s")

    @functools.partial(
        pl.kernel,
        mesh=mesh,
        out_type=jax.ShapeDtypeStruct((N_TOK * MAX_P, D), jnp.float32),
        scratch_types=[
            pltpu.VMEM((CH,), jnp.int32),
            pltpu.VMEM((CH, D), jnp.float32),
            pltpu.SemaphoreType.DMA,
        ],
    )
    def gk(table_hbm, idx_hbm, out_hbm, idx_v, rows_v, sem):
        wid = lax.axis_index("s") * 2 + lax.axis_index("c")
        base = wid * ROWS_PER_W

        def body(g, _):
            b0 = base + g * CH
            pltpu.sync_copy(idx_hbm.at[pl.ds(b0, CH)], idx_v)
            pltpu.async_copy(table_hbm.at[idx_v], rows_v, sem).wait()
            pltpu.sync_copy(rows_v, out_hbm.at[pl.ds(b0, CH)])
            return _

        lax.fori_loop(0, ROWS_PER_W // CH, body, 0)

    return gk(embed, idx_flat)


def _combine_kernel(x_ref, g_ref, ts_ref, bud_ref, out_ref, w_ref):
    ts = ts_ref[...]
    w = jax.nn.softmax(ts, axis=-1)
    ranks = lax.broadcasted_iota(jnp.int32, (1, MAX_P), 1).astype(jnp.float32)
    w_ref[...] = w * (ranks < bud_ref[...]).astype(jnp.float32)

    def body(t, _):
        x_t = x_ref[pl.ds(t, 1), :]
        g_t = g_ref[pl.ds(t * MAX_P, MAX_P), :]
        prod = lax.dot_general(x_t, g_t, (((1,), (1,)), ((), ())))
        wa = w_ref[pl.ds(t, 1), :] * jnp.tanh(prod)
        out = lax.dot_general(wa, g_t, (((1,), (0,)), ((), ())))
        out_ref[pl.ds(t, 1), :] = out + x_t
        return _

    lax.fori_loop(0, CT, body, 0)


def _combine(x, gathered, top_scores, budgets):
    nt = N_TOK // CT
    return pl.pallas_call(
        _combine_kernel,
        grid=(nt,),
        in_specs=[
            pl.BlockSpec((CT, D), lambda i: (i, 0)),
            pl.BlockSpec((CT * MAX_P, D), lambda i: (i, 0)),
            pl.BlockSpec((CT, MAX_P), lambda i: (i, 0)),
            pl.BlockSpec((CT, 1), lambda i: (i, 0)),
        ],
        out_specs=pl.BlockSpec((CT, D), lambda i: (i, 0)),
        out_shape=jax.ShapeDtypeStruct((N_TOK, D), jnp.float32),
        scratch_shapes=[pltpu.VMEM((CT, MAX_P), jnp.float32)],
    )(x, gathered, top_scores, budgets)


def kernel(x, embed, W1, b1, W2, b2, Ws1, bs1, Ws2, bs2):
    h2, budgets = _router(x, W1, b1, W2, b2, Ws1, bs1)
    Ws2p = jnp.pad(Ws2, ((0, 0), (0, POOL_PAD - POOL)))
    bs2p = jnp.pad(bs2, (0, POOL_PAD - POOL),
                   constant_values=-1e30).reshape(1, POOL_PAD)
    scores = _scores(h2, Ws2p, bs2p)
    top_scores, indices = lax.top_k(scores, MAX_P)
    gathered = _sc_gather(embed, indices.reshape(-1).astype(jnp.int32))
    return _combine(x, gathered, top_scores, budgets)


# speedup vs baseline: 13.3198x; 13.3198x over previous
import functools
import jax
import jax.numpy as jnp
from jax import lax
from jax.experimental import pallas as pl
from jax.experimental.pallas import tpu as pltpu
from jax.experimental.pallas import tpu_sc as plsc

N_TOK = 2048
D = 1024
POOL = 100000
HID = 256
MIN_P = 32
MAX_P = 128

TOK_TILE = 256
POOL_BLK = 2048
POOL_PAD = 100352
NBLK = POOL_PAD // 128
NW = 32
RW = N_TOK // NW
ROWS_PER_W = (N_TOK * MAX_P) // NW
CH = 64
CT = 16
NEG = -3.0e38


def _router_kernel(x_ref, W1_ref, b1_ref, W2_ref, b2_ref, Ws1_ref, bs1_ref,
                   h2_ref, bud_ref):
    x = x_ref[...]
    h = jnp.maximum(x @ W1_ref[...] + b1_ref[...], 0.0)
    c = jax.nn.sigmoid(h @ W2_ref[...] + b2_ref[...])
    raw = MIN_P + (MAX_P - MIN_P) * (c * c)
    bud_ref[...] = jnp.round(jnp.clip(raw, MIN_P, MAX_P))
    h2_ref[...] = jnp.maximum(x @ Ws1_ref[...] + bs1_ref[...], 0.0)


def _router(x, W1, b1, W2, b2, Ws1, bs1):
    nt = N_TOK // TOK_TILE
    return pl.pallas_call(
        _router_kernel,
        grid=(nt,),
        in_specs=[
            pl.BlockSpec((TOK_TILE, D), lambda i: (i, 0)),
            pl.BlockSpec((D, 128), lambda i: (0, 0)),
            pl.BlockSpec((1, 128), lambda i: (0, 0)),
            pl.BlockSpec((128, 1), lambda i: (0, 0)),
            pl.BlockSpec((1, 1), lambda i: (0, 0)),
            pl.BlockSpec((D, HID), lambda i: (0, 0)),
            pl.BlockSpec((1, HID), lambda i: (0, 0)),
        ],
        out_specs=[
            pl.BlockSpec((TOK_TILE, HID), lambda i: (i, 0)),
            pl.BlockSpec((TOK_TILE, 1), lambda i: (i, 0)),
        ],
        out_shape=[
            jax.ShapeDtypeStruct((N_TOK, HID), jnp.float32),
            jax.ShapeDtypeStruct((N_TOK, 1), jnp.float32),
        ],
    )(x, W1, b1.reshape(1, 128), W2, b2.reshape(1, 1), Ws1,
      bs1.reshape(1, HID))


def _scores_kernel(h2_ref, Ws2_ref, bs2_ref, out_ref, m_ref):
    s = h2_ref[...] @ Ws2_ref[...] + bs2_ref[...]
    out_ref[...] = s
    for k in range(POOL_BLK // 128):
        m_ref[:, :, k:k + 1] = jnp.max(s[:, k * 128:(k + 1) * 128], axis=1,
                                       keepdims=True)[None]


def _scores(h2, Ws2p, bs2p):
    nt = N_TOK // TOK_TILE
    np_ = POOL_PAD // POOL_BLK
    nb = POOL_BLK // 128
    return pl.pallas_call(
        _scores_kernel,
        grid=(nt, np_),
        in_specs=[
            pl.BlockSpec((TOK_TILE, HID), lambda i, j: (i, 0)),
            pl.BlockSpec((HID, POOL_BLK), lambda i, j: (0, j)),
            pl.BlockSpec((1, POOL_BLK), lambda i, j: (0, j)),
        ],
        out_specs=[
            pl.BlockSpec((TOK_TILE, POOL_BLK), lambda i, j: (i, j)),
            pl.BlockSpec((1, TOK_TILE, nb), lambda i, j: (j, i, 0)),
        ],
        out_shape=[
            jax.ShapeDtypeStruct((N_TOK, POOL_PAD), jnp.float32),
            jax.ShapeDtypeStruct((POOL_PAD // POOL_BLK, N_TOK, nb),
                                 jnp.float32),
        ],
    )(h2, Ws2p, bs2p)


def _topblocks_kernel(m_ref, bg_ref, bms_ref):
    T = m_ref.shape[0]
    bms_ref[...] = m_ref[...]
    iota = lax.broadcasted_iota(jnp.int32, (T, NBLK), 1)
    kio = lax.broadcasted_iota(jnp.int32, (T, MAX_P), 1)
    rowbase = (pl.program_id(0) * T
               + lax.broadcasted_iota(jnp.int32, (T, 1), 0)) * NBLK

    def body(k, acc):
        bm = bms_ref[...]
        m = jnp.max(bm, axis=1, keepdims=True)
        b = jnp.min(jnp.where(bm >= m, iota, NBLK), axis=1, keepdims=True)
        bms_ref[...] = jnp.where(iota == b, NEG, bm)
        return acc + jnp.where(kio == k, b + rowbase, 0)

    acc = lax.fori_loop(0, MAX_P, body,
                        jnp.zeros((T, MAX_P), jnp.int32))
    bg_ref[...] = acc


def _topblocks(M):
    T = 256
    nt = N_TOK // T
    return pl.pallas_call(
        _topblocks_kernel,
        grid=(nt,),
        in_specs=[pl.BlockSpec((T, NBLK), lambda i: (i, 0))],
        out_specs=pl.BlockSpec((T, MAX_P), lambda i: (i, 0)),
        out_shape=jax.ShapeDtypeStruct((N_TOK, MAX_P), jnp.int32),
        scratch_shapes=[pltpu.VMEM((T, NBLK), jnp.float32)],
    )(M)


def _sc_topk(scores2_flat, bg_flat):
    mesh = plsc.VectorSubcoreMesh(core_axis_name="c", subcore_axis_name="s")

    @functools.partial(
        pl.kernel,
        mesh=mesh,
        out_type=[
            jax.ShapeDtypeStruct((N_TOK * MAX_P,), jnp.float32),
            jax.ShapeDtypeStruct((N_TOK * MAX_P,), jnp.int32),
        ],
        scratch_types=[
            pltpu.VMEM((MAX_P,), jnp.int32),
            pltpu.VMEM((MAX_P + 16,), jnp.int32),
            pltpu.VMEM((MAX_P, 128), jnp.float32),
            pltpu.VMEM((MAX_P,), jnp.float32),
            pltpu.VMEM((MAX_P,), jnp.float32),
            pltpu.VMEM((MAX_P,), jnp.int32),
            pltpu.VMEM((32,), jnp.float32),
            pltpu.VMEM((32,), jnp.int32),
            pltpu.SemaphoreType.DMA,
        ],
    )
    def tk(s2_hbm, bg_hbm, val_out, idx_out, bg_v, bgp_v, cand_v, bm_v,
           ov_v, oi_v, trf_v, tri_v, sem):
        wid = lax.axis_index("s") * 2 + lax.axis_index("c")
        lane = lax.broadcasted_iota(jnp.int32, (16,), 0)

        def tree_max(s):
            for k in (8, 4, 2, 1):
                trf_v[pl.ds(0, 16)] = s
                s = jnp.maximum(s, trf_v[pl.ds(k, 16)])
            return s[0]

        def tree_min_i(s):
            for k in (8, 4, 2, 1):
                tri_v[pl.ds(0, 16)] = s
                s = jnp.minimum(s, tri_v[pl.ds(k, 16)])
            return s[0]

        def row_body(r, _):
            row = wid * RW + r
            pltpu.sync_copy(bg_hbm.at[pl.ds(row * MAX_P, MAX_P)], bg_v)
            pltpu.async_copy(s2_hbm.at[bg_v], cand_v, sem).wait()
            for g in range(MAX_P // 16):
                bgp_v[pl.ds(g * 16, 16)] = bg_v[pl.ds(g * 16, 16)]

            def bm_body(bb, carry):
                m16 = cand_v[bb, pl.ds(0, 16)]
                for j in range(1, 8):
                    m16 = jnp.maximum(m16, cand_v[bb, pl.ds(j * 16, 16)])
                m = tree_max(m16)
                carry = jnp.where(lane == bb % 16, m, carry)

                @pl.when(bb % 16 == 15)
                def _():
                    bm_v[pl.ds((bb // 16) * 16, 16)] = carry

                return carry

            lax.fori_loop(0, MAX_P, bm_body, jnp.zeros((16,), jnp.float32))

            def ext_body(k, carry):
                ovc, oic = carry
                gm16 = bm_v[pl.ds(0, 16)]
                for j in range(1, 8):
                    gm16 = jnp.maximum(gm16, bm_v[pl.ds(j * 16, 16)])
                m = tree_max(gm16)
                mb = jnp.full((16,), m, jnp.float32)

                imin = jnp.full((16,), 99999, jnp.int32)
                for j in range(8):
                    v = bm_v[pl.ds(j * 16, 16)]
                    imin = jnp.minimum(
                        imin, jnp.where(v >= mb, lane + j * 16, 99999))
                bstar = tree_min_i(imin)

                cmin = jnp.full((16,), 99999, jnp.int32)
                for j in range(8):
                    v = cand_v[bstar, pl.ds(j * 16, 16)]
                    cmin = jnp.minimum(
                        cmin, jnp.where(v >= mb, lane + j * 16, 99999))
                col = tree_min_i(cmin)

                bgval = bgp_v[pl.ds(bstar, 16)][0]
                bid = bgval - row * NBLK
                gidx = bid * 128 + col

                ovc = jnp.where(lane == k % 16, m, ovc)
                oic = jnp.where(lane == k % 16, gidx, oic)

                @pl.when(k % 16 == 15)
                def _():
                    ov_v[pl.ds((k // 16) * 16, 16)] = ovc
                    oi_v[pl.ds((k // 16) * 16, 16)] = oic

                cq = (col // 16) * 16
                vec = cand_v[bstar, pl.ds(cq, 16)]
                cand_v[bstar, pl.ds(cq, 16)] = jnp.where(
                    lane == col - cq, NEG, vec)

                m16 = cand_v[bstar, pl.ds(0, 16)]
                for j in range(1, 8):
                    m16 = jnp.maximum(m16, cand_v[bstar, pl.ds(j * 16, 16)])
                nm = tree_max(m16)
                bq = (bstar // 16) * 16
                bv = bm_v[pl.ds(bq, 16)]
                bm_v[pl.ds(bq, 16)] = jnp.where(lane == bstar - bq, nm, bv)

                return ovc, oic

            lax.fori_loop(0, MAX_P, ext_body,
                          (jnp.zeros((16,), jnp.float32),
                           jnp.zeros((16,), jnp.int32)))

            pltpu.sync_copy(ov_v, val_out.at[pl.ds(row * MAX_P, MAX_P)])
            pltpu.sync_copy(oi_v, idx_out.at[pl.ds(row * MAX_P, MAX_P)])
            return 0

        lax.fori_loop(0, RW, row_body, 0)

    return tk(scores2_flat, bg_flat)


def _sc_gather(embed, idx_flat):
    mesh = plsc.VectorSubcoreMesh(core_axis_name="c", subcore_axis_name="s")

    @functools.partial(
        pl.kernel,
        mesh=mesh,
        out_type=jax.ShapeDtypeStruct((N_TOK * MAX_P, D), jnp.float32),
        scratch_types=[
            pltpu.VMEM((CH,), jnp.int32),
            pltpu.VMEM((CH, D), jnp.float32),
            pltpu.SemaphoreType.DMA,
        ],
    )
    def gk(table_hbm, idx_hbm, out_hbm, idx_v, rows_v, sem):
        wid = lax.axis_index("s") * 2 + lax.axis_index("c")
        base = wid * ROWS_PER_W

        def body(g, _):
            b0 = base + g * CH
            pltpu.sync_copy(idx_hbm.at[pl.ds(b0, CH)], idx_v)
            pltpu.async_copy(table_hbm.at[idx_v], rows_v, sem).wait()
            pltpu.sync_copy(rows_v, out_hbm.at[pl.ds(b0, CH)])
            return _

        lax.fori_loop(0, ROWS_PER_W // CH, body, 0)

    return gk(embed, idx_flat)


def _combine_kernel(x_ref, g_ref, ts_ref, bud_ref, out_ref, w_ref):
    ts = ts_ref[...]
    w = jax.nn.softmax(ts, axis=-1)
    ranks = lax.broadcasted_iota(jnp.int32, (1, MAX_P), 1).astype(jnp.float32)
    w_ref[...] = w * (ranks < bud_ref[...]).astype(jnp.float32)

    def body(t, _):
        x_t = x_ref[pl.ds(t, 1), :]
        g_t = g_ref[pl.ds(t * MAX_P, MAX_P), :]
        prod = lax.dot_general(x_t, g_t, (((1,), (1,)), ((), ())))
        wa = w_ref[pl.ds(t, 1), :] * jnp.tanh(prod)
        out = lax.dot_general(wa, g_t, (((1,), (0,)), ((), ())))
        out_ref[pl.ds(t, 1), :] = out + x_t
        return _

    lax.fori_loop(0, CT, body, 0)


def _combine(x, gathered, top_scores, budgets):
    nt = N_TOK // CT
    return pl.pallas_call(
        _combine_kernel,
        grid=(nt,),
        in_specs=[
            pl.BlockSpec((CT, D), lambda i: (i, 0)),
            pl.BlockSpec((CT * MAX_P, D), lambda i: (i, 0)),
            pl.BlockSpec((CT, MAX_P), lambda i: (i, 0)),
            pl.BlockSpec((CT, 1), lambda i: (i, 0)),
        ],
        out_specs=pl.BlockSpec((CT, D), lambda i: (i, 0)),
        out_shape=jax.ShapeDtypeStruct((N_TOK, D), jnp.float32),
        scratch_shapes=[pltpu.VMEM((CT, MAX_P), jnp.float32)],
    )(x, gathered, top_scores, budgets)


def kernel(x, embed, W1, b1, W2, b2, Ws1, bs1, Ws2, bs2):
    h2, budgets = _router(x, W1, b1, W2, b2, Ws1, bs1)
    Ws2p = jnp.pad(Ws2, ((0, 0), (0, POOL_PAD - POOL)))
    bs2p = jnp.pad(bs2, (0, POOL_PAD - POOL),
                   constant_values=-1e30).reshape(1, POOL_PAD)
    scores, M3 = _scores(h2, Ws2p, bs2p)
    M = jnp.transpose(M3, (1, 0, 2)).reshape(N_TOK, NBLK)
    bg = _topblocks(M)
    top_scores, indices = _sc_topk(scores.reshape(N_TOK * NBLK, 128),
                                   bg.reshape(-1))
    gathered = _sc_gather(embed, indices)
    return _combine(x, gathered, top_scores.reshape(N_TOK, MAX_P), budgets)


# speedup vs baseline: 13.6782x; 1.0269x over previous
import functools
import jax
import jax.numpy as jnp
from jax import lax
from jax.experimental import pallas as pl
from jax.experimental.pallas import tpu as pltpu
from jax.experimental.pallas import tpu_sc as plsc

N_TOK = 2048
D = 1024
POOL = 100000
HID = 256
MIN_P = 32
MAX_P = 128

TOK_TILE = 256
POOL_BLK = 2048
POOL_PAD = 100352
NBLK = POOL_PAD // 128
NW = 32
RW = N_TOK // NW
ROWS_PER_W = (N_TOK * MAX_P) // NW
CH = 32
CT = 16
NEG = -3.0e38


def _router_kernel(x_ref, W1_ref, b1_ref, W2_ref, b2_ref, Ws1_ref, bs1_ref,
                   h2_ref, bud_ref):
    x = x_ref[...]
    h = jnp.maximum(x @ W1_ref[...] + b1_ref[...], 0.0)
    c = jax.nn.sigmoid(h @ W2_ref[...] + b2_ref[...])
    raw = MIN_P + (MAX_P - MIN_P) * (c * c)
    bud_ref[...] = jnp.round(jnp.clip(raw, MIN_P, MAX_P))
    h2_ref[...] = jnp.maximum(x @ Ws1_ref[...] + bs1_ref[...], 0.0)


def _router(x, W1, b1, W2, b2, Ws1, bs1):
    nt = N_TOK // TOK_TILE
    return pl.pallas_call(
        _router_kernel,
        grid=(nt,),
        in_specs=[
            pl.BlockSpec((TOK_TILE, D), lambda i: (i, 0)),
            pl.BlockSpec((D, 128), lambda i: (0, 0)),
            pl.BlockSpec((1, 128), lambda i: (0, 0)),
            pl.BlockSpec((128, 1), lambda i: (0, 0)),
            pl.BlockSpec((1, 1), lambda i: (0, 0)),
            pl.BlockSpec((D, HID), lambda i: (0, 0)),
            pl.BlockSpec((1, HID), lambda i: (0, 0)),
        ],
        out_specs=[
            pl.BlockSpec((TOK_TILE, HID), lambda i: (i, 0)),
            pl.BlockSpec((TOK_TILE, 1), lambda i: (i, 0)),
        ],
        out_shape=[
            jax.ShapeDtypeStruct((N_TOK, HID), jnp.float32),
            jax.ShapeDtypeStruct((N_TOK, 1), jnp.float32),
        ],
    )(x, W1, b1.reshape(1, 128), W2, b2.reshape(1, 1), Ws1,
      bs1.reshape(1, HID))


def _scores_kernel(h2_ref, Ws2_ref, bs2_ref, out_ref, m_ref):
    s = h2_ref[...] @ Ws2_ref[...] + bs2_ref[...]
    out_ref[...] = s
    for k in range(POOL_BLK // 128):
        m_ref[:, :, k:k + 1] = jnp.max(s[:, k * 128:(k + 1) * 128], axis=1,
                                       keepdims=True)[None]


def _scores(h2, Ws2p, bs2p):
    nt = N_TOK // TOK_TILE
    np_ = POOL_PAD // POOL_BLK
    nb = POOL_BLK // 128
    return pl.pallas_call(
        _scores_kernel,
        grid=(nt, np_),
        in_specs=[
            pl.BlockSpec((TOK_TILE, HID), lambda i, j: (i, 0)),
            pl.BlockSpec((HID, POOL_BLK), lambda i, j: (0, j)),
            pl.BlockSpec((1, POOL_BLK), lambda i, j: (0, j)),
        ],
        out_specs=[
            pl.BlockSpec((TOK_TILE, POOL_BLK), lambda i, j: (i, j)),
            pl.BlockSpec((1, TOK_TILE, nb), lambda i, j: (j, i, 0)),
        ],
        out_shape=[
            jax.ShapeDtypeStruct((N_TOK, POOL_PAD), jnp.float32),
            jax.ShapeDtypeStruct((POOL_PAD // POOL_BLK, N_TOK, nb),
                                 jnp.float32),
        ],
    )(h2, Ws2p, bs2p)


def _topblocks_kernel(m_ref, bg_ref, bms_ref):
    T = m_ref.shape[0]
    bms_ref[...] = m_ref[...]
    iota = lax.broadcasted_iota(jnp.int32, (T, NBLK), 1)
    kio = lax.broadcasted_iota(jnp.int32, (T, MAX_P), 1)
    rowbase = (pl.program_id(0) * T
               + lax.broadcasted_iota(jnp.int32, (T, 1), 0)) * NBLK

    def body(k, acc):
        bm = bms_ref[...]
        m = jnp.max(bm, axis=1, keepdims=True)
        b = jnp.min(jnp.where(bm >= m, iota, NBLK), axis=1, keepdims=True)
        bms_ref[...] = jnp.where(iota == b, NEG, bm)
        return acc + jnp.where(kio == k, b + rowbase, 0)

    acc = lax.fori_loop(0, MAX_P, body,
                        jnp.zeros((T, MAX_P), jnp.int32))
    bg_ref[...] = acc


def _topblocks(M):
    T = 256
    nt = N_TOK // T
    return pl.pallas_call(
        _topblocks_kernel,
        grid=(nt,),
        in_specs=[pl.BlockSpec((T, NBLK), lambda i: (i, 0))],
        out_specs=pl.BlockSpec((T, MAX_P), lambda i: (i, 0)),
        out_shape=jax.ShapeDtypeStruct((N_TOK, MAX_P), jnp.int32),
        scratch_shapes=[pltpu.VMEM((T, NBLK), jnp.float32)],
    )(M)


def _sc_topk(scores2_flat, bg_flat):
    mesh = plsc.VectorSubcoreMesh(core_axis_name="c", subcore_axis_name="s")

    @functools.partial(
        pl.kernel,
        mesh=mesh,
        out_type=[
            jax.ShapeDtypeStruct((N_TOK * MAX_P,), jnp.float32),
            jax.ShapeDtypeStruct((N_TOK * MAX_P,), jnp.int32),
        ],
        scratch_types=[
            pltpu.VMEM((MAX_P,), jnp.int32),
            pltpu.VMEM((MAX_P + 16,), jnp.int32),
            pltpu.VMEM((MAX_P, 128), jnp.float32),
            pltpu.VMEM((MAX_P,), jnp.float32),
            pltpu.VMEM((MAX_P,), jnp.float32),
            pltpu.VMEM((MAX_P,), jnp.int32),
            pltpu.VMEM((32,), jnp.float32),
            pltpu.VMEM((32,), jnp.int32),
            pltpu.SemaphoreType.DMA,
        ],
    )
    def tk(s2_hbm, bg_hbm, val_out, idx_out, bg_v, bgp_v, cand_v, bm_v,
           ov_v, oi_v, trf_v, tri_v, sem):
        wid = lax.axis_index("s") * 2 + lax.axis_index("c")
        lane = lax.broadcasted_iota(jnp.int32, (16,), 0)

        def tree_max(s):
            for k in (8, 4, 2, 1):
                trf_v[pl.ds(0, 16)] = s
                s = jnp.maximum(s, trf_v[pl.ds(k, 16)])
            return s[0]

        def tree_min_i(s):
            for k in (8, 4, 2, 1):
                tri_v[pl.ds(0, 16)] = s
                s = jnp.minimum(s, tri_v[pl.ds(k, 16)])
            return s[0]

        def row_body(r, _):
            row = wid * RW + r
            pltpu.sync_copy(bg_hbm.at[pl.ds(row * MAX_P, MAX_P)], bg_v)
            pltpu.async_copy(s2_hbm.at[bg_v], cand_v, sem).wait()
            for g in range(MAX_P // 16):
                bgp_v[pl.ds(g * 16, 16)] = bg_v[pl.ds(g * 16, 16)]

            def bm_body(bb, carry):
                m16 = cand_v[bb, pl.ds(0, 16)]
                for j in range(1, 8):
                    m16 = jnp.maximum(m16, cand_v[bb, pl.ds(j * 16, 16)])
                m = tree_max(m16)
                carry = jnp.where(lane == bb % 16, m, carry)

                @pl.when(bb % 16 == 15)
                def _():
                    bm_v[pl.ds((bb // 16) * 16, 16)] = carry

                return carry

            lax.fori_loop(0, MAX_P, bm_body, jnp.zeros((16,), jnp.float32))

            def ext_body(k, carry):
                ovc, oic = carry
                gm16 = bm_v[pl.ds(0, 16)]
                for j in range(1, 8):
                    gm16 = jnp.maximum(gm16, bm_v[pl.ds(j * 16, 16)])
                m = tree_max(gm16)
                mb = jnp.full((16,), m, jnp.float32)

                imin = jnp.full((16,), 99999, jnp.int32)
                for j in range(8):
                    v = bm_v[pl.ds(j * 16, 16)]
                    imin = jnp.minimum(
                        imin, jnp.where(v >= mb, lane + j * 16, 99999))
                bstar = tree_min_i(imin)

                cmin = jnp.full((16,), 99999, jnp.int32)
                for j in range(8):
                    v = cand_v[bstar, pl.ds(j * 16, 16)]
                    cmin = jnp.minimum(
                        cmin, jnp.where(v >= mb, lane + j * 16, 99999))
                col = tree_min_i(cmin)

                bgval = bgp_v[pl.ds(bstar, 16)][0]
                bid = bgval - row * NBLK
                gidx = bid * 128 + col

                ovc = jnp.where(lane == k % 16, m, ovc)
                oic = jnp.where(lane == k % 16, gidx, oic)

                @pl.when(k % 16 == 15)
                def _():
                    ov_v[pl.ds((k // 16) * 16, 16)] = ovc
                    oi_v[pl.ds((k // 16) * 16, 16)] = oic

                cq = (col // 16) * 16
                vec = cand_v[bstar, pl.ds(cq, 16)]
                cand_v[bstar, pl.ds(cq, 16)] = jnp.where(
                    lane == col - cq, NEG, vec)

                m16 = cand_v[bstar, pl.ds(0, 16)]
                for j in range(1, 8):
                    m16 = jnp.maximum(m16, cand_v[bstar, pl.ds(j * 16, 16)])
                nm = tree_max(m16)
                bq = (bstar // 16) * 16
                bv = bm_v[pl.ds(bq, 16)]
                bm_v[pl.ds(bq, 16)] = jnp.where(lane == bstar - bq, nm, bv)

                return ovc, oic

            lax.fori_loop(0, MAX_P, ext_body,
                          (jnp.zeros((16,), jnp.float32),
                           jnp.zeros((16,), jnp.int32)))

            pltpu.sync_copy(ov_v, val_out.at[pl.ds(row * MAX_P, MAX_P)])
            pltpu.sync_copy(oi_v, idx_out.at[pl.ds(row * MAX_P, MAX_P)])
            return 0

        lax.fori_loop(0, RW, row_body, 0)

    return tk(scores2_flat, bg_flat)


def _sc_gather(embed, idx_flat):
    mesh = plsc.VectorSubcoreMesh(core_axis_name="c", subcore_axis_name="s")

    @functools.partial(
        pl.kernel,
        mesh=mesh,
        out_type=jax.ShapeDtypeStruct((N_TOK * MAX_P, D), jnp.float32),
        scratch_types=[
            pltpu.VMEM((CH,), jnp.int32),
            pltpu.VMEM((CH,), jnp.int32),
            pltpu.VMEM((CH, D), jnp.float32),
            pltpu.VMEM((CH, D), jnp.float32),
            pltpu.SemaphoreType.DMA,
            pltpu.SemaphoreType.DMA,
        ],
    )
    def gk(table_hbm, idx_hbm, out_hbm, idx_a, idx_b, rows_a, rows_b,
           sem_a, sem_b):
        wid = lax.axis_index("s") * 2 + lax.axis_index("c")
        base = wid * ROWS_PER_W
        npair = ROWS_PER_W // CH // 2

        pltpu.sync_copy(idx_hbm.at[pl.ds(base, CH)], idx_a)
        pltpu.async_copy(table_hbm.at[idx_a], rows_a, sem_a)

        def body(gp, _):
            b0 = base + (2 * gp) * CH
            b1 = b0 + CH
            pltpu.sync_copy(idx_hbm.at[pl.ds(b1, CH)], idx_b)
            pltpu.async_copy(table_hbm.at[idx_b], rows_b, sem_b)
            pltpu.make_async_copy(table_hbm.at[idx_a], rows_a, sem_a).wait()
            pltpu.sync_copy(rows_a, out_hbm.at[pl.ds(b0, CH)])

            @pl.when(gp < npair - 1)
            def _():
                pltpu.sync_copy(idx_hbm.at[pl.ds(b1 + CH, CH)], idx_a)
                pltpu.async_copy(table_hbm.at[idx_a], rows_a, sem_a)

            pltpu.make_async_copy(table_hbm.at[idx_b], rows_b, sem_b).wait()
            pltpu.sync_copy(rows_b, out_hbm.at[pl.ds(b1, CH)])
            return 0

        lax.fori_loop(0, npair, body, 0)

    return gk(embed, idx_flat)


def _combine_kernel(x_ref, g_ref, ts_ref, bud_ref, out_ref, w_ref):
    ts = ts_ref[...]
    w = jax.nn.softmax(ts, axis=-1)
    ranks = lax.broadcasted_iota(jnp.int32, (1, MAX_P), 1).astype(jnp.float32)
    w_ref[...] = w * (ranks < bud_ref[...]).astype(jnp.float32)

    def body(t, _):
        x_t = x_ref[pl.ds(t, 1), :]
        g_t = g_ref[pl.ds(t * MAX_P, MAX_P), :]
        prod = lax.dot_general(x_t, g_t, (((1,), (1,)), ((), ())))
        wa = w_ref[pl.ds(t, 1), :] * jnp.tanh(prod)
        out = lax.dot_general(wa, g_t, (((1,), (0,)), ((), ())))
        out_ref[pl.ds(t, 1), :] = out + x_t
        return _

    lax.fori_loop(0, CT, body, 0)


def _combine(x, gathered, top_scores, budgets):
    nt = N_TOK // CT
    return pl.pallas_call(
        _combine_kernel,
        grid=(nt,),
        in_specs=[
            pl.BlockSpec((CT, D), lambda i: (i, 0)),
            pl.BlockSpec((CT * MAX_P, D), lambda i: (i, 0)),
            pl.BlockSpec((CT, MAX_P), lambda i: (i, 0)),
            pl.BlockSpec((CT, 1), lambda i: (i, 0)),
        ],
        out_specs=pl.BlockSpec((CT, D), lambda i: (i, 0)),
        out_shape=jax.ShapeDtypeStruct((N_TOK, D), jnp.float32),
        scratch_shapes=[pltpu.VMEM((CT, MAX_P), jnp.float32)],
    )(x, gathered, top_scores, budgets)


def kernel(x, embed, W1, b1, W2, b2, Ws1, bs1, Ws2, bs2):
    h2, budgets = _router(x, W1, b1, W2, b2, Ws1, bs1)
    Ws2p = jnp.pad(Ws2, ((0, 0), (0, POOL_PAD - POOL)))
    bs2p = jnp.pad(bs2, (0, POOL_PAD - POOL),
                   constant_values=-1e30).reshape(1, POOL_PAD)
    scores, M3 = _scores(h2, Ws2p, bs2p)
    M = jnp.transpose(M3, (1, 0, 2)).reshape(N_TOK, NBLK)
    bg = _topblocks(M)
    top_scores, indices = _sc_topk(scores.reshape(N_TOK * NBLK, 128),
                                   bg.reshape(-1))
    gathered = _sc_gather(embed, indices)
    return _combine(x, gathered, top_scores.reshape(N_TOK, MAX_P), budgets)


# speedup vs baseline: 13.9072x; 1.0167x over previous
import functools
import jax
import jax.numpy as jnp
from jax import lax
from jax.experimental import pallas as pl
from jax.experimental.pallas import tpu as pltpu
from jax.experimental.pallas import tpu_sc as plsc

N_TOK = 2048
D = 1024
POOL = 100000
HID = 256
MIN_P = 32
MAX_P = 128

TOK_TILE = 256
POOL_BLK = 2048
POOL_PAD = 100352
NBLK = POOL_PAD // 128
NW = 32
RW = N_TOK // NW
ROWS_PER_W = (N_TOK * MAX_P) // NW
CH = 32
CT = 16
NEG = -3.0e38


def _router_kernel(x_ref, W1_ref, b1_ref, W2_ref, b2_ref, Ws1_ref, bs1_ref,
                   h2_ref, bud_ref):
    x = x_ref[...]
    h = jnp.maximum(x @ W1_ref[...] + b1_ref[...], 0.0)
    c = jax.nn.sigmoid(h @ W2_ref[...] + b2_ref[...])
    raw = MIN_P + (MAX_P - MIN_P) * (c * c)
    bud_ref[...] = jnp.round(jnp.clip(raw, MIN_P, MAX_P))
    h2_ref[...] = jnp.maximum(x @ Ws1_ref[...] + bs1_ref[...], 0.0)


def _router(x, W1, b1, W2, b2, Ws1, bs1):
    nt = N_TOK // TOK_TILE
    return pl.pallas_call(
        _router_kernel,
        grid=(nt,),
        in_specs=[
            pl.BlockSpec((TOK_TILE, D), lambda i: (i, 0)),
            pl.BlockSpec((D, 128), lambda i: (0, 0)),
            pl.BlockSpec((1, 128), lambda i: (0, 0)),
            pl.BlockSpec((128, 1), lambda i: (0, 0)),
            pl.BlockSpec((1, 1), lambda i: (0, 0)),
            pl.BlockSpec((D, HID), lambda i: (0, 0)),
            pl.BlockSpec((1, HID), lambda i: (0, 0)),
        ],
        out_specs=[
            pl.BlockSpec((TOK_TILE, HID), lambda i: (i, 0)),
            pl.BlockSpec((TOK_TILE, 1), lambda i: (i, 0)),
        ],
        out_shape=[
            jax.ShapeDtypeStruct((N_TOK, HID), jnp.float32),
            jax.ShapeDtypeStruct((N_TOK, 1), jnp.float32),
        ],
    )(x, W1, b1.reshape(1, 128), W2, b2.reshape(1, 1), Ws1,
      bs1.reshape(1, HID))


def _scores_kernel(h2_ref, Ws2_ref, bs2_ref, out_ref, m_ref):
    s = h2_ref[...] @ Ws2_ref[...] + bs2_ref[...]
    out_ref[...] = s
    for k in range(POOL_BLK // 128):
        m_ref[:, :, k:k + 1] = jnp.max(s[:, k * 128:(k + 1) * 128], axis=1,
                                       keepdims=True)[None]


def _scores(h2, Ws2p, bs2p):
    nt = N_TOK // TOK_TILE
    np_ = POOL_PAD // POOL_BLK
    nb = POOL_BLK // 128
    return pl.pallas_call(
        _scores_kernel,
        grid=(nt, np_),
        in_specs=[
            pl.BlockSpec((TOK_TILE, HID), lambda i, j: (i, 0)),
            pl.BlockSpec((HID, POOL_BLK), lambda i, j: (0, j)),
            pl.BlockSpec((1, POOL_BLK), lambda i, j: (0, j)),
        ],
        out_specs=[
            pl.BlockSpec((TOK_TILE, POOL_BLK), lambda i, j: (i, j)),
            pl.BlockSpec((1, TOK_TILE, nb), lambda i, j: (j, i, 0)),
        ],
        out_shape=[
            jax.ShapeDtypeStruct((N_TOK, POOL_PAD), jnp.float32),
            jax.ShapeDtypeStruct((POOL_PAD // POOL_BLK, N_TOK, nb),
                                 jnp.float32),
        ],
    )(h2, Ws2p, bs2p)


def _topblocks_kernel(m_ref, bg_ref, bms_ref):
    T = m_ref.shape[0]
    bms_ref[...] = m_ref[...]
    iota = lax.broadcasted_iota(jnp.int32, (T, NBLK), 1)
    kio = lax.broadcasted_iota(jnp.int32, (T, MAX_P), 1)
    rowbase = (pl.program_id(0) * T
               + lax.broadcasted_iota(jnp.int32, (T, 1), 0)) * NBLK

    def body(k, acc):
        bm = bms_ref[...]
        m = jnp.max(bm, axis=1, keepdims=True)
        b = jnp.min(jnp.where(bm >= m, iota, NBLK), axis=1, keepdims=True)
        bms_ref[...] = jnp.where(iota == b, NEG, bm)
        return acc + jnp.where(kio == k, b + rowbase, 0)

    acc = lax.fori_loop(0, MAX_P, body,
                        jnp.zeros((T, MAX_P), jnp.int32))
    bg_ref[...] = acc


def _topblocks(M):
    T = 256
    nt = N_TOK // T
    return pl.pallas_call(
        _topblocks_kernel,
        grid=(nt,),
        in_specs=[pl.BlockSpec((T, NBLK), lambda i: (i, 0))],
        out_specs=pl.BlockSpec((T, MAX_P), lambda i: (i, 0)),
        out_shape=jax.ShapeDtypeStruct((N_TOK, MAX_P), jnp.int32),
        scratch_shapes=[pltpu.VMEM((T, NBLK), jnp.float32)],
    )(M)


def _sc_topk(scores2_flat, bg_flat):
    mesh = plsc.VectorSubcoreMesh(core_axis_name="c", subcore_axis_name="s")

    @functools.partial(
        pl.kernel,
        mesh=mesh,
        out_type=[
            jax.ShapeDtypeStruct((N_TOK * MAX_P,), jnp.float32),
            jax.ShapeDtypeStruct((N_TOK * MAX_P,), jnp.int32),
        ],
        scratch_types=[
            pltpu.VMEM((MAX_P,), jnp.int32),
            pltpu.VMEM((MAX_P,), jnp.int32),
            pltpu.VMEM((MAX_P + 16,), jnp.int32),
            pltpu.VMEM((MAX_P, 128), jnp.float32),
            pltpu.VMEM((MAX_P, 128), jnp.float32),
            pltpu.VMEM((MAX_P,), jnp.float32),
            pltpu.VMEM((MAX_P,), jnp.float32),
            pltpu.VMEM((MAX_P,), jnp.int32),
            pltpu.VMEM((32,), jnp.float32),
            pltpu.VMEM((32,), jnp.int32),
            pltpu.SemaphoreType.DMA,
            pltpu.SemaphoreType.DMA,
        ],
    )
    def tk(s2_hbm, bg_hbm, val_out, idx_out, bg_a, bg_b, bgp_v, cand_a,
           cand_b, bm_v, ov_v, oi_v, trf_v, tri_v, sem_a, sem_b):
        wid = lax.axis_index("s") * 2 + lax.axis_index("c")
        lane = lax.broadcasted_iota(jnp.int32, (16,), 0)

        def tree_max(s):
            for k in (8, 4, 2, 1):
                trf_v[pl.ds(0, 16)] = s
                s = jnp.maximum(s, trf_v[pl.ds(k, 16)])
            return s[0]

        def tree_min_i(s):
            for k in (8, 4, 2, 1):
                tri_v[pl.ds(0, 16)] = s
                s = jnp.minimum(s, tri_v[pl.ds(k, 16)])
            return s[0]

        def process(row, bg_ref, cand_v):
            for g in range(MAX_P // 16):
                bgp_v[pl.ds(g * 16, 16)] = bg_ref[pl.ds(g * 16, 16)]

            def bm_body(bb, carry):
                m16 = cand_v[bb, pl.ds(0, 16)]
                for j in range(1, 8):
                    m16 = jnp.maximum(m16, cand_v[bb, pl.ds(j * 16, 16)])
                m = tree_max(m16)
                carry = jnp.where(lane == bb % 16, m, carry)

                @pl.when(bb % 16 == 15)
                def _():
                    bm_v[pl.ds((bb // 16) * 16, 16)] = carry

                return carry

            lax.fori_loop(0, MAX_P, bm_body, jnp.zeros((16,), jnp.float32))

            def ext_body(k, carry):
                ovc, oic = carry
                gm16 = bm_v[pl.ds(0, 16)]
                for j in range(1, 8):
                    gm16 = jnp.maximum(gm16, bm_v[pl.ds(j * 16, 16)])
                m = tree_max(gm16)
                mb = jnp.full((16,), m, jnp.float32)

                imin = jnp.full((16,), 99999, jnp.int32)
                for j in range(8):
                    v = bm_v[pl.ds(j * 16, 16)]
                    imin = jnp.minimum(
                        imin, jnp.where(v >= mb, lane + j * 16, 99999))
                bstar = tree_min_i(imin)

                cmin = jnp.full((16,), 99999, jnp.int32)
                for j in range(8):
                    v = cand_v[bstar, pl.ds(j * 16, 16)]
                    cmin = jnp.minimum(
                        cmin, jnp.where(v >= mb, lane + j * 16, 99999))
                col = tree_min_i(cmin)

                bgval = bgp_v[pl.ds(bstar, 16)][0]
                bid = bgval - row * NBLK
                gidx = bid * 128 + col

                ovc = jnp.where(lane == k % 16, m, ovc)
                oic = jnp.where(lane == k % 16, gidx, oic)

                @pl.when(k % 16 == 15)
                def _():
                    ov_v[pl.ds((k // 16) * 16, 16)] = ovc
                    oi_v[pl.ds((k // 16) * 16, 16)] = oic

                cq = (col // 16) * 16
                vec = cand_v[bstar, pl.ds(cq, 16)]
                cand_v[bstar, pl.ds(cq, 16)] = jnp.where(
                    lane == col - cq, NEG, vec)

                m16 = cand_v[bstar, pl.ds(0, 16)]
                for j in range(1, 8):
                    m16 = jnp.maximum(m16, cand_v[bstar, pl.ds(j * 16, 16)])
                nm = tree_max(m16)
                bq = (bstar // 16) * 16
                bv = bm_v[pl.ds(bq, 16)]
                bm_v[pl.ds(bq, 16)] = jnp.where(lane == bstar - bq, nm, bv)

                return ovc, oic

            lax.fori_loop(0, MAX_P, ext_body,
                          (jnp.zeros((16,), jnp.float32),
                           jnp.zeros((16,), jnp.int32)))

            pltpu.sync_copy(ov_v, val_out.at[pl.ds(row * MAX_P, MAX_P)])
            pltpu.sync_copy(oi_v, idx_out.at[pl.ds(row * MAX_P, MAX_P)])

        base = wid * RW
        pltpu.sync_copy(bg_hbm.at[pl.ds(base * MAX_P, MAX_P)], bg_a)
        pltpu.async_copy(s2_hbm.at[bg_a], cand_a, sem_a)

        def pair_body(gp, _):
            row0 = base + 2 * gp
            row1 = row0 + 1
            pltpu.sync_copy(bg_hbm.at[pl.ds(row1 * MAX_P, MAX_P)], bg_b)
            pltpu.async_copy(s2_hbm.at[bg_b], cand_b, sem_b)
            pltpu.make_async_copy(s2_hbm.at[bg_a], cand_a, sem_a).wait()
            process(row0, bg_a, cand_a)

            @pl.when(gp < RW // 2 - 1)
            def _():
                pltpu.sync_copy(bg_hbm.at[pl.ds((row0 + 2) * MAX_P, MAX_P)],
                                bg_a)
                pltpu.async_copy(s2_hbm.at[bg_a], cand_a, sem_a)

            pltpu.make_async_copy(s2_hbm.at[bg_b], cand_b, sem_b).wait()
            process(row1, bg_b, cand_b)
            return 0

        lax.fori_loop(0, RW // 2, pair_body, 0)

    return tk(scores2_flat, bg_flat)


def _sc_gather(embed, idx_flat):
    mesh = plsc.VectorSubcoreMesh(core_axis_name="c", subcore_axis_name="s")

    @functools.partial(
        pl.kernel,
        mesh=mesh,
        out_type=jax.ShapeDtypeStruct((N_TOK * MAX_P, D), jnp.float32),
        scratch_types=[
            pltpu.VMEM((CH,), jnp.int32),
            pltpu.VMEM((CH,), jnp.int32),
            pltpu.VMEM((CH, D), jnp.float32),
            pltpu.VMEM((CH, D), jnp.float32),
            pltpu.SemaphoreType.DMA,
            pltpu.SemaphoreType.DMA,
        ],
    )
    def gk(table_hbm, idx_hbm, out_hbm, idx_a, idx_b, rows_a, rows_b,
           sem_a, sem_b):
        wid = lax.axis_index("s") * 2 + lax.axis_index("c")
        base = wid * ROWS_PER_W
        npair = ROWS_PER_W // CH // 2

        pltpu.sync_copy(idx_hbm.at[pl.ds(base, CH)], idx_a)
        pltpu.async_copy(table_hbm.at[idx_a], rows_a, sem_a)

        def body(gp, _):
            b0 = base + (2 * gp) * CH
            b1 = b0 + CH
            pltpu.sync_copy(idx_hbm.at[pl.ds(b1, CH)], idx_b)
            pltpu.async_copy(table_hbm.at[idx_b], rows_b, sem_b)
            pltpu.make_async_copy(table_hbm.at[idx_a], rows_a, sem_a).wait()
            pltpu.sync_copy(rows_a, out_hbm.at[pl.ds(b0, CH)])

            @pl.when(gp < npair - 1)
            def _():
                pltpu.sync_copy(idx_hbm.at[pl.ds(b1 + CH, CH)], idx_a)
                pltpu.async_copy(table_hbm.at[idx_a], rows_a, sem_a)

            pltpu.make_async_copy(table_hbm.at[idx_b], rows_b, sem_b).wait()
            pltpu.sync_copy(rows_b, out_hbm.at[pl.ds(b1, CH)])
            return 0

        lax.fori_loop(0, npair, body, 0)

    return gk(embed, idx_flat)


def _combine_kernel(x_ref, g_ref, ts_ref, bud_ref, out_ref, w_ref):
    ts = ts_ref[...]
    w = jax.nn.softmax(ts, axis=-1)
    ranks = lax.broadcasted_iota(jnp.int32, (1, MAX_P), 1).astype(jnp.float32)
    w_ref[...] = w * (ranks < bud_ref[...]).astype(jnp.float32)

    def body(t, _):
        x_t = x_ref[pl.ds(t, 1), :]
        g_t = g_ref[pl.ds(t * MAX_P, MAX_P), :]
        prod = lax.dot_general(x_t, g_t, (((1,), (1,)), ((), ())))
        wa = w_ref[pl.ds(t, 1), :] * jnp.tanh(prod)
        out = lax.dot_general(wa, g_t, (((1,), (0,)), ((), ())))
        out_ref[pl.ds(t, 1), :] = out + x_t
        return _

    lax.fori_loop(0, CT, body, 0)


def _combine(x, gathered, top_scores, budgets):
    nt = N_TOK // CT
    return pl.pallas_call(
        _combine_kernel,
        grid=(nt,),
        in_specs=[
            pl.BlockSpec((CT, D), lambda i: (i, 0)),
            pl.BlockSpec((CT * MAX_P, D), lambda i: (i, 0)),
            pl.BlockSpec((CT, MAX_P), lambda i: (i, 0)),
            pl.BlockSpec((CT, 1), lambda i: (i, 0)),
        ],
        out_specs=pl.BlockSpec((CT, D), lambda i: (i, 0)),
        out_shape=jax.ShapeDtypeStruct((N_TOK, D), jnp.float32),
        scratch_shapes=[pltpu.VMEM((CT, MAX_P), jnp.float32)],
    )(x, gathered, top_scores, budgets)


def kernel(x, embed, W1, b1, W2, b2, Ws1, bs1, Ws2, bs2):
    h2, budgets = _router(x, W1, b1, W2, b2, Ws1, bs1)
    Ws2p = jnp.pad(Ws2, ((0, 0), (0, POOL_PAD - POOL)))
    bs2p = jnp.pad(bs2, (0, POOL_PAD - POOL),
                   constant_values=-1e30).reshape(1, POOL_PAD)
    scores, M3 = _scores(h2, Ws2p, bs2p)
    M = jnp.transpose(M3, (1, 0, 2)).reshape(N_TOK, NBLK)
    bg = _topblocks(M)
    top_scores, indices = _sc_topk(scores.reshape(N_TOK * NBLK, 128),
                                   bg.reshape(-1))
    gathered = _sc_gather(embed, indices)
    return _combine(x, gathered, top_scores.reshape(N_TOK, MAX_P), budgets)


# speedup vs baseline: 14.9523x; 1.0751x over previous
import functools
import jax
import jax.numpy as jnp
from jax import lax
from jax.experimental import pallas as pl
from jax.experimental.pallas import tpu as pltpu
from jax.experimental.pallas import tpu_sc as plsc

N_TOK = 2048
D = 1024
POOL = 100000
HID = 256
MIN_P = 32
MAX_P = 128

TOK_TILE = 256
POOL_BLK = 2048
POOL_PAD = 100352
NBLK = POOL_PAD // 128
NW = 32
RW = N_TOK // NW
ROWS_PER_W = (N_TOK * MAX_P) // NW
CH = 32
CT = 16
NEG = -3.0e38


def _router_kernel(x_ref, W1_ref, b1_ref, W2_ref, b2_ref, Ws1_ref, bs1_ref,
                   h2_ref, bud_ref):
    x = x_ref[...]
    h = jnp.maximum(x @ W1_ref[...] + b1_ref[...], 0.0)
    c = jax.nn.sigmoid(h @ W2_ref[...] + b2_ref[...])
    raw = MIN_P + (MAX_P - MIN_P) * (c * c)
    bud_ref[...] = jnp.round(jnp.clip(raw, MIN_P, MAX_P))
    h2_ref[...] = jnp.maximum(x @ Ws1_ref[...] + bs1_ref[...], 0.0)


def _router(x, W1, b1, W2, b2, Ws1, bs1):
    nt = N_TOK // TOK_TILE
    return pl.pallas_call(
        _router_kernel,
        grid=(nt,),
        in_specs=[
            pl.BlockSpec((TOK_TILE, D), lambda i: (i, 0)),
            pl.BlockSpec((D, 128), lambda i: (0, 0)),
            pl.BlockSpec((1, 128), lambda i: (0, 0)),
            pl.BlockSpec((128, 1), lambda i: (0, 0)),
            pl.BlockSpec((1, 1), lambda i: (0, 0)),
            pl.BlockSpec((D, HID), lambda i: (0, 0)),
            pl.BlockSpec((1, HID), lambda i: (0, 0)),
        ],
        out_specs=[
            pl.BlockSpec((TOK_TILE, HID), lambda i: (i, 0)),
            pl.BlockSpec((TOK_TILE, 1), lambda i: (i, 0)),
        ],
        out_shape=[
            jax.ShapeDtypeStruct((N_TOK, HID), jnp.float32),
            jax.ShapeDtypeStruct((N_TOK, 1), jnp.float32),
        ],
    )(x, W1, b1.reshape(1, 128), W2, b2.reshape(1, 1), Ws1,
      bs1.reshape(1, HID))


def _scores_kernel(h2_ref, Ws2_ref, bs2_ref, out_ref, m_ref):
    s = h2_ref[...] @ Ws2_ref[...] + bs2_ref[...]
    out_ref[...] = s
    for k in range(POOL_BLK // 128):
        m_ref[:, :, k:k + 1] = jnp.max(s[:, k * 128:(k + 1) * 128], axis=1,
                                       keepdims=True)[None]


def _scores(h2, Ws2p, bs2p):
    nt = N_TOK // TOK_TILE
    np_ = POOL_PAD // POOL_BLK
    nb = POOL_BLK // 128
    return pl.pallas_call(
        _scores_kernel,
        grid=(nt, np_),
        in_specs=[
            pl.BlockSpec((TOK_TILE, HID), lambda i, j: (i, 0)),
            pl.BlockSpec((HID, POOL_BLK), lambda i, j: (0, j)),
            pl.BlockSpec((1, POOL_BLK), lambda i, j: (0, j)),
        ],
        out_specs=[
            pl.BlockSpec((TOK_TILE, POOL_BLK), lambda i, j: (i, j)),
            pl.BlockSpec((1, TOK_TILE, nb), lambda i, j: (j, i, 0)),
        ],
        out_shape=[
            jax.ShapeDtypeStruct((N_TOK, POOL_PAD), jnp.float32),
            jax.ShapeDtypeStruct((POOL_PAD // POOL_BLK, N_TOK, nb),
                                 jnp.float32),
        ],
    )(h2, Ws2p, bs2p)


def _topblocks_kernel(m_ref, bg_ref, bms_ref):
    T = m_ref.shape[0]
    bms_ref[...] = m_ref[...]
    iota = lax.broadcasted_iota(jnp.int32, (T, NBLK), 1)
    kio = lax.broadcasted_iota(jnp.int32, (T, MAX_P), 1)
    rowbase = (pl.program_id(0) * T
               + lax.broadcasted_iota(jnp.int32, (T, 1), 0)) * NBLK

    def body(k, acc):
        bm = bms_ref[...]
        m = jnp.max(bm, axis=1, keepdims=True)
        b = jnp.min(jnp.where(bm >= m, iota, NBLK), axis=1, keepdims=True)
        bms_ref[...] = jnp.where(iota == b, NEG, bm)
        return acc + jnp.where(kio == k, b + rowbase, 0)

    acc = lax.fori_loop(0, MAX_P, body,
                        jnp.zeros((T, MAX_P), jnp.int32))
    bg_ref[...] = acc


def _topblocks(M):
    T = 256
    nt = N_TOK // T
    return pl.pallas_call(
        _topblocks_kernel,
        grid=(nt,),
        in_specs=[pl.BlockSpec((T, NBLK), lambda i: (i, 0))],
        out_specs=pl.BlockSpec((T, MAX_P), lambda i: (i, 0)),
        out_shape=jax.ShapeDtypeStruct((N_TOK, MAX_P), jnp.int32),
        scratch_shapes=[pltpu.VMEM((T, NBLK), jnp.float32)],
    )(M)


def _sc_topk(scores2_flat, bg_flat):
    mesh = plsc.VectorSubcoreMesh(core_axis_name="c", subcore_axis_name="s")

    @functools.partial(
        pl.kernel,
        mesh=mesh,
        out_type=[
            jax.ShapeDtypeStruct((N_TOK * MAX_P,), jnp.float32),
            jax.ShapeDtypeStruct((N_TOK * MAX_P,), jnp.int32),
        ],
        scratch_types=[
            pltpu.VMEM((MAX_P,), jnp.int32),
            pltpu.VMEM((MAX_P,), jnp.int32),
            pltpu.VMEM((MAX_P + 16,), jnp.int32),
            pltpu.VMEM((MAX_P, 128), jnp.float32),
            pltpu.VMEM((MAX_P, 128), jnp.float32),
            pltpu.VMEM((MAX_P,), jnp.float32),
            pltpu.VMEM((MAX_P,), jnp.float32),
            pltpu.VMEM((MAX_P,), jnp.int32),
            pltpu.VMEM((32,), jnp.float32),
            pltpu.VMEM((32,), jnp.int32),
            pltpu.SemaphoreType.DMA,
            pltpu.SemaphoreType.DMA,
        ],
    )
    def tk(s2_hbm, bg_hbm, val_out, idx_out, bg_a, bg_b, bgp_v, cand_a,
           cand_b, bm_v, ov_v, oi_v, trf_v, tri_v, sem_a, sem_b):
        wid = lax.axis_index("s") * 2 + lax.axis_index("c")
        lane = lax.broadcasted_iota(jnp.int32, (16,), 0)

        def tree_max(s):
            for k in (8, 4, 2, 1):
                trf_v[pl.ds(0, 16)] = s
                s = jnp.maximum(s, trf_v[pl.ds(k, 16)])
            return s[0]

        def tree_min_i(s):
            for k in (8, 4, 2, 1):
                tri_v[pl.ds(0, 16)] = s
                s = jnp.minimum(s, tri_v[pl.ds(k, 16)])
            return s[0]

        def process(row, bg_ref, cand_v):
            for g in range(MAX_P // 16):
                bgp_v[pl.ds(g * 16, 16)] = bg_ref[pl.ds(g * 16, 16)]

            def bm_body(bb, carry):
                m16 = cand_v[bb, pl.ds(0, 16)]
                for j in range(1, 8):
                    m16 = jnp.maximum(m16, cand_v[bb, pl.ds(j * 16, 16)])
                m = tree_max(m16)
                carry = jnp.where(lane == bb % 16, m, carry)

                @pl.when(bb % 16 == 15)
                def _():
                    bm_v[pl.ds((bb // 16) * 16, 16)] = carry

                return carry

            lax.fori_loop(0, MAX_P, bm_body, jnp.zeros((16,), jnp.float32))

            def ext_body(k, carry):
                ovc, oic = carry
                m16 = bm_v[pl.ds(0, 16)]
                i16 = lane
                for j in range(1, 8):
                    v = bm_v[pl.ds(j * 16, 16)]
                    i16 = jnp.where(v > m16, lane + j * 16, i16)
                    m16 = jnp.maximum(m16, v)
                s, i = m16, i16
                for st in (8, 4, 2, 1):
                    trf_v[pl.ds(0, 16)] = s
                    tri_v[pl.ds(0, 16)] = i
                    s2 = trf_v[pl.ds(st, 16)]
                    i2 = tri_v[pl.ds(st, 16)]
                    i = jnp.where(s2 > s, i2,
                                  jnp.where(s2 == s, jnp.minimum(i, i2), i))
                    s = jnp.maximum(s, s2)
                m = s[0]
                bstar = i[0]
                mb = jnp.full((16,), m, jnp.float32)

                negv = jnp.full((16,), NEG, jnp.float32)
                cmin = jnp.full((16,), 99999, jnp.int32)
                m2 = negv
                for j in range(8):
                    v = cand_v[bstar, pl.ds(j * 16, 16)]
                    hit = v >= mb
                    cmin = jnp.minimum(
                        cmin, jnp.where(hit, lane + j * 16, 99999))
                    m2 = jnp.maximum(m2, jnp.where(hit, negv, v))
                col = tree_min_i(cmin)
                nm = tree_max(m2)

                bgval = bgp_v[pl.ds(bstar, 16)][0]
                bid = bgval - row * NBLK
                gidx = bid * 128 + col

                ovc = jnp.where(lane == k % 16, m, ovc)
                oic = jnp.where(lane == k % 16, gidx, oic)

                @pl.when(k % 16 == 15)
                def _():
                    ov_v[pl.ds((k // 16) * 16, 16)] = ovc
                    oi_v[pl.ds((k // 16) * 16, 16)] = oic

                cq = (col // 16) * 16
                vec = cand_v[bstar, pl.ds(cq, 16)]
                cand_v[bstar, pl.ds(cq, 16)] = jnp.where(
                    lane == col - cq, NEG, vec)

                bq = (bstar // 16) * 16
                bv = bm_v[pl.ds(bq, 16)]
                bm_v[pl.ds(bq, 16)] = jnp.where(lane == bstar - bq, nm, bv)

                return ovc, oic

            lax.fori_loop(0, MAX_P, ext_body,
                          (jnp.zeros((16,), jnp.float32),
                           jnp.zeros((16,), jnp.int32)))

            pltpu.sync_copy(ov_v, val_out.at[pl.ds(row * MAX_P, MAX_P)])
            pltpu.sync_copy(oi_v, idx_out.at[pl.ds(row * MAX_P, MAX_P)])

        base = wid * RW
        pltpu.sync_copy(bg_hbm.at[pl.ds(base * MAX_P, MAX_P)], bg_a)
        pltpu.async_copy(s2_hbm.at[bg_a], cand_a, sem_a)

        def pair_body(gp, _):
            row0 = base + 2 * gp
            row1 = row0 + 1
            pltpu.sync_copy(bg_hbm.at[pl.ds(row1 * MAX_P, MAX_P)], bg_b)
            pltpu.async_copy(s2_hbm.at[bg_b], cand_b, sem_b)
            pltpu.make_async_copy(s2_hbm.at[bg_a], cand_a, sem_a).wait()
            process(row0, bg_a, cand_a)

            @pl.when(gp < RW // 2 - 1)
            def _():
                pltpu.sync_copy(bg_hbm.at[pl.ds((row0 + 2) * MAX_P, MAX_P)],
                                bg_a)
                pltpu.async_copy(s2_hbm.at[bg_a], cand_a, sem_a)

            pltpu.make_async_copy(s2_hbm.at[bg_b], cand_b, sem_b).wait()
            process(row1, bg_b, cand_b)
            return 0

        lax.fori_loop(0, RW // 2, pair_body, 0)

    return tk(scores2_flat, bg_flat)


def _sc_gather(embed, idx_flat):
    mesh = plsc.VectorSubcoreMesh(core_axis_name="c", subcore_axis_name="s")

    @functools.partial(
        pl.kernel,
        mesh=mesh,
        out_type=jax.ShapeDtypeStruct((N_TOK * MAX_P, D), jnp.float32),
        scratch_types=[
            pltpu.VMEM((CH,), jnp.int32),
            pltpu.VMEM((CH,), jnp.int32),
            pltpu.VMEM((CH, D), jnp.float32),
            pltpu.VMEM((CH, D), jnp.float32),
            pltpu.SemaphoreType.DMA,
            pltpu.SemaphoreType.DMA,
        ],
    )
    def gk(table_hbm, idx_hbm, out_hbm, idx_a, idx_b, rows_a, rows_b,
           sem_a, sem_b):
        wid = lax.axis_index("s") * 2 + lax.axis_index("c")
        base = wid * ROWS_PER_W
        npair = ROWS_PER_W // CH // 2

        pltpu.sync_copy(idx_hbm.at[pl.ds(base, CH)], idx_a)
        pltpu.async_copy(table_hbm.at[idx_a], rows_a, sem_a)

        def body(gp, _):
            b0 = base + (2 * gp) * CH
            b1 = b0 + CH
            pltpu.sync_copy(idx_hbm.at[pl.ds(b1, CH)], idx_b)
            pltpu.async_copy(table_hbm.at[idx_b], rows_b, sem_b)
            pltpu.make_async_copy(table_hbm.at[idx_a], rows_a, sem_a).wait()
            pltpu.sync_copy(rows_a, out_hbm.at[pl.ds(b0, CH)])

            @pl.when(gp < npair - 1)
            def _():
                pltpu.sync_copy(idx_hbm.at[pl.ds(b1 + CH, CH)], idx_a)
                pltpu.async_copy(table_hbm.at[idx_a], rows_a, sem_a)

            pltpu.make_async_copy(table_hbm.at[idx_b], rows_b, sem_b).wait()
            pltpu.sync_copy(rows_b, out_hbm.at[pl.ds(b1, CH)])
            return 0

        lax.fori_loop(0, npair, body, 0)

    return gk(embed, idx_flat)


def _combine_kernel(x_ref, g_ref, ts_ref, bud_ref, out_ref, w_ref):
    ts = ts_ref[...]
    w = jax.nn.softmax(ts, axis=-1)
    ranks = lax.broadcasted_iota(jnp.int32, (1, MAX_P), 1).astype(jnp.float32)
    w_ref[...] = w * (ranks < bud_ref[...]).astype(jnp.float32)

    def body(t, _):
        x_t = x_ref[pl.ds(t, 1), :]
        g_t = g_ref[pl.ds(t * MAX_P, MAX_P), :]
        prod = lax.dot_general(x_t, g_t, (((1,), (1,)), ((), ())))
        wa = w_ref[pl.ds(t, 1), :] * jnp.tanh(prod)
        out = lax.dot_general(wa, g_t, (((1,), (0,)), ((), ())))
        out_ref[pl.ds(t, 1), :] = out + x_t
        return _

    lax.fori_loop(0, CT, body, 0)


def _combine(x, gathered, top_scores, budgets):
    nt = N_TOK // CT
    return pl.pallas_call(
        _combine_kernel,
        grid=(nt,),
        in_specs=[
            pl.BlockSpec((CT, D), lambda i: (i, 0)),
            pl.BlockSpec((CT * MAX_P, D), lambda i: (i, 0)),
            pl.BlockSpec((CT, MAX_P), lambda i: (i, 0)),
            pl.BlockSpec((CT, 1), lambda i: (i, 0)),
        ],
        out_specs=pl.BlockSpec((CT, D), lambda i: (i, 0)),
        out_shape=jax.ShapeDtypeStruct((N_TOK, D), jnp.float32),
        scratch_shapes=[pltpu.VMEM((CT, MAX_P), jnp.float32)],
    )(x, gathered, top_scores, budgets)


def kernel(x, embed, W1, b1, W2, b2, Ws1, bs1, Ws2, bs2):
    h2, budgets = _router(x, W1, b1, W2, b2, Ws1, bs1)
    Ws2p = jnp.pad(Ws2, ((0, 0), (0, POOL_PAD - POOL)))
    bs2p = jnp.pad(bs2, (0, POOL_PAD - POOL),
                   constant_values=-1e30).reshape(1, POOL_PAD)
    scores, M3 = _scores(h2, Ws2p, bs2p)
    M = jnp.transpose(M3, (1, 0, 2)).reshape(N_TOK, NBLK)
    bg = _topblocks(M)
    top_scores, indices = _sc_topk(scores.reshape(N_TOK * NBLK, 128),
                                   bg.reshape(-1))
    gathered = _sc_gather(embed, indices)
    return _combine(x, gathered, top_scores.reshape(N_TOK, MAX_P), budgets)
